# baseline (speedup 1.0000x reference)
import functools

import jax
import jax.numpy as jnp
from jax import lax
from jax.experimental import pallas as pl
from jax.experimental.pallas import tpu as pltpu
from jax.experimental.pallas import tpu_sc as plsc

_BATCH = 16384

_info = plsc.get_sparse_core_info()
_NC = _info.num_cores
_NS = _info.num_subcores
_NW = _NC * _NS
_B_PER_W = _BATCH // _NW

_mesh = plsc.VectorSubcoreMesh(core_axis_name="c", subcore_axis_name="s")


_NCHUNK = 4
_CHUNK = _B_PER_W // _NCHUNK


@functools.partial(
    pl.kernel,
    mesh=_mesh,
    out_type=jax.ShapeDtypeStruct((_BATCH,), jnp.float32),
    scratch_types=[
        pltpu.VMEM((_B_PER_W,), jnp.int32),
        *[pltpu.VMEM((_CHUNK,), jnp.float32) for _ in range(_NCHUNK)],
        *[pltpu.SemaphoreType.DMA for _ in range(_NCHUNK)],
    ],
)
def _gather_sc(items_hbm, scores_hbm, out_hbm, idx_v, *vals_and_sems):
    vals = vals_and_sems[:_NCHUNK]
    sems = vals_and_sems[_NCHUNK:]
    wid = lax.axis_index("s") * _NC + lax.axis_index("c")
    base = wid * _B_PER_W
    pltpu.sync_copy(items_hbm.at[pl.ds(base, _B_PER_W)], idx_v)
    gathers = [
        pltpu.async_copy(
            scores_hbm.at[idx_v.at[pl.ds(j * _CHUNK, _CHUNK)]], vals[j], sems[j]
        )
        for j in range(_NCHUNK)
    ]
    outs = []
    for j in range(_NCHUNK):
        gathers[j].wait()
        outs.append(
            pltpu.async_copy(
                vals[j], out_hbm.at[pl.ds(base + j * _CHUNK, _CHUNK)], sems[j]
            )
        )
    for o in outs:
        o.wait()


def kernel(users, items, scores):
    del users
    return _gather_sc(items, scores)


# speedup vs baseline: 1.0134x; 1.0134x over previous
import functools

import jax
import jax.numpy as jnp
from jax import lax
from jax.experimental import pallas as pl
from jax.experimental.pallas import tpu as pltpu
from jax.experimental.pallas import tpu_sc as plsc

_BATCH = 16384

_info = plsc.get_sparse_core_info()
_NC = _info.num_cores
_NS = _info.num_subcores
_NW = _NC * _NS
_B_PER_W = _BATCH // _NW

_mesh = plsc.VectorSubcoreMesh(core_axis_name="c", subcore_axis_name="s")


_NCHUNK = 2
_CHUNK = _B_PER_W // _NCHUNK


@functools.partial(
    pl.kernel,
    mesh=_mesh,
    out_type=jax.ShapeDtypeStruct((_BATCH,), jnp.float32),
    scratch_types=[
        *[pltpu.VMEM((_CHUNK,), jnp.int32) for _ in range(_NCHUNK)],
        *[pltpu.VMEM((_CHUNK,), jnp.float32) for _ in range(_NCHUNK)],
        *[pltpu.SemaphoreType.DMA for _ in range(_NCHUNK)],
    ],
)
def _gather_sc(items_hbm, scores_hbm, out_hbm, *scratch):
    idxs = scratch[:_NCHUNK]
    vals = scratch[_NCHUNK : 2 * _NCHUNK]
    sems = scratch[2 * _NCHUNK :]
    wid = lax.axis_index("s") * _NC + lax.axis_index("c")
    base = wid * _B_PER_W
    loads = [
        pltpu.async_copy(
            items_hbm.at[pl.ds(base + j * _CHUNK, _CHUNK)], idxs[j], sems[j]
        )
        for j in range(_NCHUNK)
    ]
    gathers = []
    for j in range(_NCHUNK):
        loads[j].wait()
        gathers.append(pltpu.async_copy(scores_hbm.at[idxs[j]], vals[j], sems[j]))
    outs = []
    for j in range(_NCHUNK):
        gathers[j].wait()
        outs.append(
            pltpu.async_copy(
                vals[j], out_hbm.at[pl.ds(base + j * _CHUNK, _CHUNK)], sems[j]
            )
        )
    for o in outs:
        o.wait()


def kernel(users, items, scores):
    del users
    return _gather_sc(items, scores)
